# baseline (speedup 1.0000x reference)
import jax
import jax.numpy as jnp
from jax.experimental import pallas as pl
from jax.experimental.pallas import tpu as pltpu

_SIZE = 32000
_SMOOTHING = 0.1
_CONFIDENCE = 1.0 - _SMOOTHING
_EPS = _SMOOTHING / (_SIZE - 2)

_ROWS = 8192
_BLOCK_R = 64
_NBUF = 4
_NB = _ROWS // _BLOCK_R


def _smooth_kernel(tgt_ref, out_ref, scratch, sems):
    i = pl.program_id(0)
    slot = jax.lax.rem(i, _NBUF)

    @pl.when(i >= _NBUF)
    def _wait_prev():
        pltpu.make_async_copy(
            scratch.at[slot],
            out_ref.at[pl.ds((i - _NBUF) * _BLOCK_R, _BLOCK_R), :],
            sems.at[slot],
        ).wait()

    tgt = tgt_ref[0, 0, :]
    cols = jax.lax.broadcasted_iota(jnp.int32, (_BLOCK_R, _SIZE), 1)
    scratch[slot, :, :] = jnp.where(
        cols == tgt[:, None],
        jnp.float32(_CONFIDENCE),
        jnp.float32(_EPS),
    )

    pltpu.make_async_copy(
        scratch.at[slot],
        out_ref.at[pl.ds(i * _BLOCK_R, _BLOCK_R), :],
        sems.at[slot],
    ).start()

    @pl.when(i == _NB - 1)
    def _drain():
        for j in range(_NB - _NBUF, _NB):
            s = j % _NBUF
            pltpu.make_async_copy(
                scratch.at[s],
                out_ref.at[pl.ds(j * _BLOCK_R, _BLOCK_R), :],
                sems.at[s],
            ).wait()


def kernel(target):
    tgt3 = target.astype(jnp.int32).reshape(_NB, 1, _BLOCK_R)
    out = pl.pallas_call(
        _smooth_kernel,
        grid=(_NB,),
        in_specs=[pl.BlockSpec((1, 1, _BLOCK_R), lambda i: (i, 0, 0))],
        out_specs=pl.BlockSpec(memory_space=pltpu.HBM),
        out_shape=jax.ShapeDtypeStruct((_ROWS, _SIZE), jnp.float32),
        scratch_shapes=[
            pltpu.VMEM((_NBUF, _BLOCK_R, _SIZE), jnp.float32),
            pltpu.SemaphoreType.DMA((_NBUF,)),
        ],
    )(tgt3)
    return out
